# baseline (speedup 1.0000x reference)
import jax
import jax.numpy as jnp
from jax import lax
from jax.experimental import pallas as pl
from jax.experimental.pallas import tpu as pltpu
from jax.experimental.pallas import tpu_sc as plsc

_E = 16
_K = 2
_T = 32768
_NS = 16
_NW = 32
_C = _T // _NS
_NV = _C // 16
_W = _T // _NW
_NB = _W // 128


def _body(ids_hbm, scores_hbm, scores_out, tok_out, cnt_out, xch_out,
          inter_out, ids_v, scoresall_v, srt_v, loc_v, pos2d, qv,
          slin_v, tlin_v, hist_v, loffself_v, off_v, exc_v, stage_v,
          counts_all, loffall_v, countsT_v, sem):
  sid = lax.axis_index("s")
  cid = lax.axis_index("c")
  base = sid * _C
  pltpu.sync_copy(ids_hbm.at[pl.ds(base, _C)], ids_v)
  pltpu.sync_copy(scores_hbm, scoresall_v)

  hist_v[...] = jnp.zeros((_E,), jnp.int32)
  iota16 = lax.iota(jnp.int32, 16)
  for j in range(_NV):
    ids16 = ids_v[pl.ds(j * 16, 16)]
    occ, last = plsc.scan_count(ids16)
    b = plsc.load_gather(hist_v, [ids16])
    loc_v[pl.ds(j * 16, 16)] = b + occ - 1
    plsc.store_scatter(hist_v, [ids16], b + occ, mask=last)

  h = hist_v[...]
  loffself_v[...] = plsc.cumsum(h) - h
  for j in range(_NV):
    ids16 = ids_v[pl.ds(j * 16, 16)]
    dl = plsc.load_gather(loffself_v, [ids16]) + loc_v[pl.ds(j * 16, 16)]
    plsc.store_scatter(srt_v, [dl], base + j * 16 + iota16)
  pltpu.sync_copy(srt_v, inter_out.at[pl.ds(base, _C)])

  pltpu.sync_copy(hist_v, xch_out.at[cid * _NS + sid])
  plsc.subcore_barrier()
  pltpu.sync_copy(xch_out.at[pl.ds(cid * _NS, _NS)], counts_all)

  zeros16 = jnp.zeros((_E,), jnp.int32)
  totals = zeros16
  for w in range(_NS):
    row = counts_all[w, :]
    totals = totals + row
    loffall_v[w, :] = plsc.cumsum(row) - row
  offsets = plsc.cumsum(totals) - totals
  off_v[...] = offsets
  for e in range(_E):
    countsT_v[e, :] = plsc.load_gather(
        counts_all, [iota16, jnp.full((16,), e, jnp.int32)])

  @pl.when(jnp.logical_and(sid == 0, cid == 0))
  def _():
    stage_v[...] = totals.astype(jnp.float32)
    pltpu.sync_copy(stage_v, cnt_out)

  wid = cid * _NS + sid
  o_lo = wid * _W
  e0 = jnp.sum((offsets <= o_lo).astype(jnp.int32), axis=0) - 1
  e0s = zeros16 + e0
  off_e0 = plsc.load_gather(off_v, [e0s])[0]
  r0 = o_lo - off_e0
  colcnt = plsc.load_gather(countsT_v, [e0s, iota16])
  exc = plsc.cumsum(colcnt) - colcnt
  exc_v[...] = exc
  w0 = jnp.sum((exc <= r0).astype(jnp.int32), axis=0) - 1
  g0 = off_e0 + plsc.load_gather(exc_v, [zeros16 + w0])[0]

  for bb in range(_NB):
    for k in range(128 // 16):
      pos2d[bb, pl.ds(k * 16, 16)] = zeros16

  def walk_body(state):
    e, w, g, dst = state
    ws = zeros16 + w
    es = zeros16 + e
    cnt = plsc.load_gather(counts_all, [ws, es])[0]
    lo = jnp.maximum(g, o_lo)
    n = jnp.maximum(jnp.minimum(g + cnt, o_lo + _W) - lo, 0)
    src = w * _C + plsc.load_gather(loffall_v, [ws, es])[0] + (lo - g)

    def fill(k, _):
      t16 = dst + k * 16 + iota16
      vals = src + k * 16 + iota16
      mask = (k * 16 + iota16) < n
      plsc.store_scatter(
          pos2d, [lax.shift_right_logical(t16, 7),
                  lax.bitwise_and(t16, 127)], vals, mask=mask)
      return 0

    lax.fori_loop(0, (n + 15) // 16, fill, 0)
    w1 = w + 1
    wrap = (w1 == _NS).astype(jnp.int32)
    return (e + wrap, w1 * (1 - wrap), g + cnt, dst + n)

  lax.while_loop(lambda s: s[3] < _W, walk_body,
                 (e0, w0, g0, jnp.int32(0)))

  copies = [pltpu.async_copy(inter_out.at[pos2d.at[bb]], qv.at[bb], sem)
            for bb in range(_NB)]
  for d in copies:
    d.wait()

  for j in range(_W // 16):
    bb, k = divmod(j, 128 // 16)
    q16 = qv[bb, pl.ds(k * 16, 16)]
    q16 = jnp.minimum(jnp.maximum(q16, 0), _T - 1)
    slin_v[pl.ds(j * 16, 16)] = plsc.load_gather(scoresall_v, [q16])
    tlin_v[pl.ds(j * 16, 16)] = lax.shift_right_logical(q16, 1)
  pltpu.sync_copy(slin_v, scores_out.at[pl.ds(o_lo, _W)])
  pltpu.sync_copy(tlin_v, tok_out.at[pl.ds(o_lo, _W)])


@jax.jit
def kernel(top_scores, selected_experts_indices):
  ids = selected_experts_indices.reshape(-1)
  scores = top_scores.reshape(-1)
  mesh = plsc.VectorSubcoreMesh(
      core_axis_name="c", subcore_axis_name="s", num_cores=2)
  scores_sorted, tok_sorted, counts, _, _ = pl.kernel(
      _body,
      out_type=(
          jax.ShapeDtypeStruct((_T,), jnp.float32),
          jax.ShapeDtypeStruct((_T,), jnp.int32),
          jax.ShapeDtypeStruct((_E,), jnp.float32),
          jax.ShapeDtypeStruct((_NW, _E), jnp.int32),
          jax.ShapeDtypeStruct((_T,), jnp.int32),
      ),
      mesh=mesh,
      compiler_params=pltpu.CompilerParams(needs_layout_passes=False),
      scratch_types=[
          pltpu.VMEM((_C,), jnp.int32),
          pltpu.VMEM((_T,), jnp.float32),
          pltpu.VMEM((_C,), jnp.int32),
          pltpu.VMEM((_C,), jnp.int32),
          pltpu.VMEM((_NB, 128), jnp.int32),
          pltpu.VMEM((_NB, 128), jnp.int32),
          pltpu.VMEM((_W,), jnp.float32),
          pltpu.VMEM((_W,), jnp.int32),
          pltpu.VMEM((_E,), jnp.int32),
          pltpu.VMEM((_E,), jnp.int32),
          pltpu.VMEM((_E,), jnp.int32),
          pltpu.VMEM((_NS,), jnp.int32),
          pltpu.VMEM((_E,), jnp.float32),
          pltpu.VMEM((_NS, _E), jnp.int32),
          pltpu.VMEM((_NS, _E), jnp.int32),
          pltpu.VMEM((_E, _NS), jnp.int32),
          pltpu.SemaphoreType.DMA,
      ],
  )(ids, scores)
  return scores_sorted, tok_sorted, counts


# speedup vs baseline: 1.0917x; 1.0917x over previous
import jax
import jax.numpy as jnp
from jax import lax
from jax.experimental import pallas as pl
from jax.experimental.pallas import tpu as pltpu
from jax.experimental.pallas import tpu_sc as plsc

_E = 16
_K = 2
_T = 32768
_NS = 16
_NW = 32
_C = _T // _NS
_NV = _C // 16
_W = _T // _NW
_NB = _W // 128


def _body(ids_hbm, scores_hbm, scores_out, tok_out, cnt_out, xch_out,
          inter_out, ids_v, scoresall_v, srt_v, loc_v, pos2d, qv,
          slin_v, tlin_v, hist_v, loffself_v, off_v, exc_v, stage_v,
          counts_all, loffall_v, countsT_v, sem, sem2):
  sid = lax.axis_index("s")
  cid = lax.axis_index("c")
  base = sid * _C
  pltpu.sync_copy(ids_hbm.at[pl.ds(base, _C)], ids_v)
  scores_copy = pltpu.async_copy(scores_hbm, scoresall_v, sem2)

  hist_v[...] = jnp.zeros((_E,), jnp.int32)
  iota16 = lax.iota(jnp.int32, 16)
  for j in range(_NV):
    ids16 = ids_v[pl.ds(j * 16, 16)]
    occ, last = plsc.scan_count(ids16)
    b = plsc.load_gather(hist_v, [ids16])
    loc_v[pl.ds(j * 16, 16)] = b + occ - 1
    plsc.store_scatter(hist_v, [ids16], b + occ, mask=last)

  h = hist_v[...]
  loffself_v[...] = plsc.cumsum(h) - h
  for j in range(_NV):
    ids16 = ids_v[pl.ds(j * 16, 16)]
    dl = plsc.load_gather(loffself_v, [ids16]) + loc_v[pl.ds(j * 16, 16)]
    plsc.store_scatter(srt_v, [dl], base + j * 16 + iota16)
  pltpu.sync_copy(srt_v, inter_out.at[pl.ds(base, _C)])

  pltpu.sync_copy(hist_v, xch_out.at[cid * _NS + sid])
  plsc.subcore_barrier()
  pltpu.sync_copy(xch_out.at[pl.ds(cid * _NS, _NS)], counts_all)

  zeros16 = jnp.zeros((_E,), jnp.int32)
  totals = zeros16
  for w in range(_NS):
    row = counts_all[w, :]
    totals = totals + row
    loffall_v[w, :] = plsc.cumsum(row) - row
  offsets = plsc.cumsum(totals) - totals
  off_v[...] = offsets
  for e in range(_E):
    countsT_v[e, :] = plsc.load_gather(
        counts_all, [iota16, jnp.full((16,), e, jnp.int32)])

  @pl.when(jnp.logical_and(sid == 0, cid == 0))
  def _():
    stage_v[...] = totals.astype(jnp.float32)
    pltpu.sync_copy(stage_v, cnt_out)

  wid = cid * _NS + sid
  o_lo = wid * _W
  e0 = jnp.sum((offsets <= o_lo).astype(jnp.int32), axis=0) - 1
  e0s = zeros16 + e0
  off_e0 = plsc.load_gather(off_v, [e0s])[0]
  r0 = o_lo - off_e0
  colcnt = plsc.load_gather(countsT_v, [e0s, iota16])
  exc = plsc.cumsum(colcnt) - colcnt
  exc_v[...] = exc
  w0 = jnp.sum((exc <= r0).astype(jnp.int32), axis=0) - 1
  g0 = off_e0 + plsc.load_gather(exc_v, [zeros16 + w0])[0]

  for bb in range(_NB):
    for k in range(128 // 16):
      pos2d[bb, pl.ds(k * 16, 16)] = zeros16

  def walk_body(state):
    e, w, g, dst = state
    ws = zeros16 + w
    es = zeros16 + e
    cnt = plsc.load_gather(counts_all, [ws, es])[0]
    lo = jnp.maximum(g, o_lo)
    n = jnp.maximum(jnp.minimum(g + cnt, o_lo + _W) - lo, 0)
    src = w * _C + plsc.load_gather(loffall_v, [ws, es])[0] + (lo - g)

    def fill(k, _):
      t16 = dst + k * 16 + iota16
      vals = src + k * 16 + iota16
      mask = (k * 16 + iota16) < n
      plsc.store_scatter(
          pos2d, [lax.shift_right_logical(t16, 7),
                  lax.bitwise_and(t16, 127)], vals, mask=mask)
      return 0

    lax.fori_loop(0, (n + 15) // 16, fill, 0)
    w1 = w + 1
    wrap = (w1 == _NS).astype(jnp.int32)
    return (e + wrap, w1 * (1 - wrap), g + cnt, dst + n)

  lax.while_loop(lambda s: s[3] < _W, walk_body,
                 (e0, w0, g0, jnp.int32(0)))

  copies = [pltpu.async_copy(inter_out.at[pos2d.at[bb]], qv.at[bb], sem)
            for bb in range(_NB)]
  for d in copies:
    d.wait()

  scores_copy.wait()
  for j in range(_W // 16):
    bb, k = divmod(j, 128 // 16)
    q16 = qv[bb, pl.ds(k * 16, 16)]
    q16 = jnp.minimum(jnp.maximum(q16, 0), _T - 1)
    slin_v[pl.ds(j * 16, 16)] = plsc.load_gather(scoresall_v, [q16])
    tlin_v[pl.ds(j * 16, 16)] = lax.shift_right_logical(q16, 1)
  pltpu.sync_copy(slin_v, scores_out.at[pl.ds(o_lo, _W)])
  pltpu.sync_copy(tlin_v, tok_out.at[pl.ds(o_lo, _W)])


@jax.jit
def kernel(top_scores, selected_experts_indices):
  ids = selected_experts_indices.reshape(-1)
  scores = top_scores.reshape(-1)
  mesh = plsc.VectorSubcoreMesh(
      core_axis_name="c", subcore_axis_name="s", num_cores=2)
  scores_sorted, tok_sorted, counts, _, _ = pl.kernel(
      _body,
      out_type=(
          jax.ShapeDtypeStruct((_T,), jnp.float32),
          jax.ShapeDtypeStruct((_T,), jnp.int32),
          jax.ShapeDtypeStruct((_E,), jnp.float32),
          jax.ShapeDtypeStruct((_NW, _E), jnp.int32),
          jax.ShapeDtypeStruct((_T,), jnp.int32),
      ),
      mesh=mesh,
      compiler_params=pltpu.CompilerParams(needs_layout_passes=False),
      scratch_types=[
          pltpu.VMEM((_C,), jnp.int32),
          pltpu.VMEM((_T,), jnp.float32),
          pltpu.VMEM((_C,), jnp.int32),
          pltpu.VMEM((_C,), jnp.int32),
          pltpu.VMEM((_NB, 128), jnp.int32),
          pltpu.VMEM((_NB, 128), jnp.int32),
          pltpu.VMEM((_W,), jnp.float32),
          pltpu.VMEM((_W,), jnp.int32),
          pltpu.VMEM((_E,), jnp.int32),
          pltpu.VMEM((_E,), jnp.int32),
          pltpu.VMEM((_E,), jnp.int32),
          pltpu.VMEM((_NS,), jnp.int32),
          pltpu.VMEM((_E,), jnp.float32),
          pltpu.VMEM((_NS, _E), jnp.int32),
          pltpu.VMEM((_NS, _E), jnp.int32),
          pltpu.VMEM((_E, _NS), jnp.int32),
          pltpu.SemaphoreType.DMA,
          pltpu.SemaphoreType.DMA,
      ],
  )(ids, scores)
  return scores_sorted, tok_sorted, counts
